# baseline (speedup 1.0000x reference)
import functools

import jax
import jax.numpy as jnp
from jax import lax
from jax.experimental import pallas as pl
from jax.experimental.pallas import tpu as pltpu
from jax.experimental.pallas import tpu_sc as plsc

_ROWS = 16384
_COLS = 200
_D = 128
_N = _ROWS * _COLS
_NC = 2
_NS = 16
_NW = _NC * _NS
_G = 128
_NBUF = 4
_L = 16
_ROWS_PER_W = _N // _NW
_PHASES = 2
_PH_ROWS = _ROWS_PER_W // _PHASES
_PH_STEPS = _PH_ROWS // _G
_IDX_ROWS = _N // _G


def _sc_gather(table, idx):
    mesh = plsc.VectorSubcoreMesh(core_axis_name="c", subcore_axis_name="s")

    @functools.partial(
        pl.kernel,
        mesh=mesh,
        out_type=jax.ShapeDtypeStruct((_IDX_ROWS, _G, _D), jnp.float32),
        scratch_types=[
            pltpu.VMEM_SHARED((3, _D), jnp.float32),
            pltpu.VMEM((3, _D), jnp.float32),
            pltpu.VMEM((_PH_STEPS, _G), jnp.int32),
            pltpu.VMEM((_NBUF, _G, _D), jnp.float32),
            pltpu.SemaphoreType.DMA((_NBUF,)),
            pltpu.SemaphoreType.DMA((_NBUF,)),
        ],
    )
    def k(table_hbm, idx_hbm, out_hbm, tab_s, tab_v, idx_v, rows_v, gsem, ssem):
        wid = lax.axis_index("s") * _NC + lax.axis_index("c")
        base_row = wid * (_ROWS_PER_W // _G)

        @pl.when(lax.axis_index("s") == 0)
        def _():
            pltpu.sync_copy(table_hbm, tab_s)

        pltpu.sync_copy(table_hbm, tab_v)
        plsc.subcore_barrier()

        def run_phase(ph, carry):
            ph_row = base_row + ph * _PH_STEPS
            pltpu.sync_copy(idx_hbm.at[pl.ds(ph_row, _PH_STEPS)], idx_v)

            trow = [
                [tab_v[r, pl.ds(kk * _L, _L)] for kk in range(_D // _L)]
                for r in range(3)
            ]

            def fire(g, b):
                pltpu.make_async_copy(
                    tab_s.at[idx_v.at[g]], rows_v.at[b], gsem.at[b]
                ).start()

            def wait_gather(g, b):
                pltpu.make_async_copy(
                    tab_s.at[idx_v.at[g]], rows_v.at[b], gsem.at[b]
                ).wait()

            def scatter_start(g, b):
                pltpu.make_async_copy(
                    rows_v.at[b], out_hbm.at[ph_row + g], ssem.at[b]
                ).start()

            def scatter_wait(g, b):
                pltpu.make_async_copy(
                    rows_v.at[b], out_hbm.at[ph_row + g], ssem.at[b]
                ).wait()

            def tec_fill(g, b):
                def grp(c16, carry3):
                    iv = idx_v[g, pl.ds(c16 * _L, _L)]
                    for l in range(_L):
                        c = c16 * _L + l
                        sf = iv[l].astype(jnp.float32)
                        w0 = jnp.full((_L,), (1.0 - sf) * (2.0 - sf) * 0.5)
                        w1 = jnp.full((_L,), sf * (2.0 - sf))
                        w2 = jnp.full((_L,), sf * (sf - 1.0) * 0.5)
                        for kk in range(_D // _L):
                            val = (w0 * trow[0][kk] + w1 * trow[1][kk]
                                   + w2 * trow[2][kk])
                            rows_v[b, c, pl.ds(kk * _L, _L)] = val
                    return carry3

                lax.fori_loop(0, _G // _L, grp, 0)

            fire(0, 0)

            def body(q, carry2):
                g0 = _NBUF * q
                g1 = g0 + 1
                g2 = g0 + 2
                g3 = g0 + 3

                wait_gather(g0, 0)
                scatter_start(g0, 0)

                @pl.when(q >= 1)
                def _():
                    scatter_wait(g2 - _NBUF, 2)
                fire(g2, 2)

                @pl.when(q >= 1)
                def _():
                    scatter_wait(g1 - _NBUF, 1)
                tec_fill(g1, 1)
                scatter_start(g1, 1)

                wait_gather(g2, 2)
                scatter_start(g2, 2)

                @pl.when(g0 + _NBUF < _PH_STEPS)
                def _():
                    scatter_wait(g0, 0)
                    fire(g0 + _NBUF, 0)

                @pl.when(q >= 1)
                def _():
                    scatter_wait(g3 - _NBUF, 3)
                tec_fill(g3, 3)
                scatter_start(g3, 3)
                return carry2

            lax.fori_loop(0, _PH_STEPS // _NBUF, body, 0)
            for i in range(_NBUF):
                g = _PH_STEPS - _NBUF + i
                scatter_wait(g, g % _NBUF)
            return carry

        lax.fori_loop(0, _PHASES, run_phase, 0)

    return k(table, idx)


def kernel(regime, table):
    idx = regime.astype(jnp.int32).reshape(_IDX_ROWS, _G)
    out = _sc_gather(table, idx)
    return out.reshape(_ROWS, _COLS, _D)
